# baseline (speedup 1.0000x reference)
import jax
import jax.numpy as jnp
from jax.experimental import pallas as pl
from jax.experimental.pallas import tpu as pltpu

_T = 256
_B = 1024
_C = 192
_REMAIN = 64
_BT = 8
_STEPS = _REMAIN // _BT
_IDX_ROWS = _T // _STEPS


def _shuffle_kernel(p_ref, out_ref, idx_ref):
    i = pl.program_id(0)
    for k in range(_BT):
        out_ref[k, :, :] = p_ref[_BT - 1 - k, :, :]
    row = i * _IDX_ROWS + jax.lax.broadcasted_iota(
        jnp.int32, (_IDX_ROWS, _B), 0)
    idx_ref[...] = (_T - 1) - row


def kernel(patches):
    out, idx = pl.pallas_call(
        _shuffle_kernel,
        grid=(_STEPS,),
        in_specs=[
            pl.BlockSpec((_BT, _B, _C), lambda i: (_T // _BT - 1 - i, 0, 0)),
        ],
        out_specs=[
            pl.BlockSpec((_BT, _B, _C), lambda i: (i, 0, 0)),
            pl.BlockSpec((_IDX_ROWS, _B), lambda i: (i, 0)),
        ],
        compiler_params=pltpu.CompilerParams(
            dimension_semantics=("parallel",)),
        out_shape=[
            jax.ShapeDtypeStruct((_REMAIN, _B, _C), patches.dtype),
            jax.ShapeDtypeStruct((_T, _B), jnp.int32),
        ],
    )(patches)
    return (out, idx, idx)
